# baseline (speedup 1.0000x reference)
import functools

import jax
import jax.numpy as jnp
from jax import lax
from jax.experimental import pallas as pl
from jax.experimental.pallas import tpu as pltpu
from jax.experimental.pallas import tpu_sc as plsc

def _sc_mesh():
    return dict(
        mesh=plsc.VectorSubcoreMesh(core_axis_name="c", subcore_axis_name="s"),
        compiler_params=pltpu.CompilerParams(needs_layout_passes=False),
    )
NW = 32

B = 32
N = 512
H = 128
HALF = 64
EMB_DIM = 300
MAXDEG = 20
SAMPLE = 10
LAYERS = 3
T = 1024
M = B * N
CT = 128
NTC = T // CT


_VR = 2000


def _premul_body(x_ref, wf_ref, wb_ref, bf_ref, bb_ref, tf_ref, tb_ref):
    x = x_ref[...]
    tf_ref[...] = jnp.dot(x, wf_ref[...], preferred_element_type=jnp.float32) + bf_ref[...]
    tb_ref[...] = jnp.dot(x, wb_ref[...], preferred_element_type=jnp.float32) + bb_ref[...]


def _premul(emb, Wf, Wb, bf, bb):
    V = emb.shape[0]
    grid = (V // _VR,)
    return pl.pallas_call(
        _premul_body,
        grid=grid,
        in_specs=[
            pl.BlockSpec((_VR, EMB_DIM), lambda i: (i, 0)),
            pl.BlockSpec((EMB_DIM, 4 * HALF), lambda i: (0, 0)),
            pl.BlockSpec((EMB_DIM, 4 * HALF), lambda i: (0, 0)),
            pl.BlockSpec((1, 4 * HALF), lambda i: (0, 0)),
            pl.BlockSpec((1, 4 * HALF), lambda i: (0, 0)),
        ],
        out_specs=[
            pl.BlockSpec((_VR, 4 * HALF), lambda i: (i, 0)),
            pl.BlockSpec((_VR, 4 * HALF), lambda i: (i, 0)),
        ],
        out_shape=[
            jax.ShapeDtypeStruct((V, 4 * HALF), jnp.float32),
            jax.ShapeDtypeStruct((V, 4 * HALF), jnp.float32),
        ],
    )(emb, Wf, Wb, bf, bb)


def _lstm_body(gf_ref, gb_ref, w2_ref,
               hsf_ref, hsb_ref, htf_ref, htb_ref,
               h2_ref, cf_ref, cb_ref):
    i = pl.program_id(0)

    @pl.when(i == 0)
    def _():
        h2_ref[...] = jnp.zeros_like(h2_ref)
        cf_ref[...] = jnp.zeros_like(cf_ref)
        cb_ref[...] = jnp.zeros_like(cb_ref)

    def gates_math(g, c):
        sg = jax.nn.sigmoid(g[:, 0:2 * HALF])
        gg = jnp.tanh(g[:, 2 * HALF:3 * HALF])
        og = jax.nn.sigmoid(g[:, 3 * HALF:4 * HALF])
        c2 = sg[:, HALF:2 * HALF] * c + sg[:, 0:HALF] * gg
        h2 = og * jnp.tanh(c2)
        return h2, c2

    def step(t, _):
        tr = CT - 1 - t
        g2 = (jnp.concatenate([gf_ref[t], gb_ref[tr]], axis=-1)
              + jnp.dot(h2_ref[...], w2_ref[...], preferred_element_type=jnp.float32))
        h2f, c2f = gates_math(g2[:, 0:4 * HALF], cf_ref[...])
        h2b, c2b = gates_math(g2[:, 4 * HALF:8 * HALF], cb_ref[...])
        cf_ref[...] = c2f
        cb_ref[...] = c2b
        h2_ref[:, 0:HALF] = h2f
        h2_ref[:, HALF:2 * HALF] = h2b
        hsf_ref[t] = h2f
        hsb_ref[tr] = h2b
        return 0

    lax.fori_loop(0, CT, step, 0)

    @pl.when(i == NTC - 1)
    def _():
        htf_ref[...] = h2_ref[:, 0:HALF]
        htb_ref[...] = h2_ref[:, HALF:2 * HALF]


def _bilstm(gf, gb, W2):
    return pl.pallas_call(
        _lstm_body,
        grid=(NTC,),
        in_specs=[
            pl.BlockSpec((CT, B, 4 * HALF), lambda i: (i, 0, 0)),
            pl.BlockSpec((CT, B, 4 * HALF), lambda i: (NTC - 1 - i, 0, 0)),
            pl.BlockSpec((2 * HALF, 8 * HALF), lambda i: (0, 0)),
        ],
        out_specs=[
            pl.BlockSpec((CT, B, HALF), lambda i: (i, 0, 0)),
            pl.BlockSpec((CT, B, HALF), lambda i: (NTC - 1 - i, 0, 0)),
            pl.BlockSpec((B, HALF), lambda i: (0, 0)),
            pl.BlockSpec((B, HALF), lambda i: (0, 0)),
        ],
        out_shape=[
            jax.ShapeDtypeStruct((T, B, HALF), jnp.float32),
            jax.ShapeDtypeStruct((T, B, HALF), jnp.float32),
            jax.ShapeDtypeStruct((B, HALF), jnp.float32),
            jax.ShapeDtypeStruct((B, HALF), jnp.float32),
        ],
        scratch_shapes=[
            pltpu.VMEM((B, 2 * HALF), jnp.float32),
            pltpu.VMEM((B, HALF), jnp.float32),
            pltpu.VMEM((B, HALF), jnp.float32),
        ],
    )(gf, gb, W2)


def _agg_body(hf_ref, sf_ref, df_ref, hb_ref, sb_ref, db_ref,
              w1f_ref, w2f_ref, bf_ref, w1b_ref, w2b_ref, bb_ref,
              of_ref, ob_ref):
    nf = sf_ref[...] * df_ref[...]
    nb = sb_ref[...] * db_ref[...]
    of_ref[...] = jax.nn.relu(
        jnp.dot(hf_ref[...], w1f_ref[...], preferred_element_type=jnp.float32)
        + jnp.dot(nf, w2f_ref[...], preferred_element_type=jnp.float32) + bf_ref[...])
    ob_ref[...] = jax.nn.relu(
        jnp.dot(hb_ref[...], w1b_ref[...], preferred_element_type=jnp.float32)
        + jnp.dot(nb, w2b_ref[...], preferred_element_type=jnp.float32) + bb_ref[...])


def _aggregate(hf, sumf, invdf, hb, sumb, invdb, w1f, w2f, bf, w1b, w2b, bb):
    R = 2048
    grid = (M // R,)
    row = lambda i: (i, 0)
    fixed = lambda i: (0, 0)
    return pl.pallas_call(
        _agg_body,
        grid=grid,
        in_specs=[
            pl.BlockSpec((R, H), row), pl.BlockSpec((R, H), row), pl.BlockSpec((R, 1), row),
            pl.BlockSpec((R, H), row), pl.BlockSpec((R, H), row), pl.BlockSpec((R, 1), row),
            pl.BlockSpec((H, H), fixed), pl.BlockSpec((H, H), fixed), pl.BlockSpec((1, H), fixed),
            pl.BlockSpec((H, H), fixed), pl.BlockSpec((H, H), fixed), pl.BlockSpec((1, H), fixed),
        ],
        out_specs=[pl.BlockSpec((R, H), row), pl.BlockSpec((R, H), row)],
        out_shape=[
            jax.ShapeDtypeStruct((M, H), jnp.float32),
            jax.ShapeDtypeStruct((M, H), jnp.float32),
        ],
    )(hf, sumf, invdf, hb, sumb, invdb, w1f, w2f, bf, w1b, w2b, bb)


def _gates_body(tf_hbm, tb_hbm, ids_hbm, gf_hbm, gb_hbm, idx_v, rows_v, sem):
    wid = lax.axis_index("s") * 2 + lax.axis_index("c")
    rpw = (T * B) // NW
    base = wid * rpw
    pltpu.sync_copy(ids_hbm.at[pl.ds(base, rpw)], idx_v)

    def chunk(c, _):
        cb = c * 128
        idx = idx_v.at[pl.ds(cb, 128)]
        pltpu.async_copy(tf_hbm.at[idx], rows_v, sem).wait()
        pltpu.sync_copy(rows_v, gf_hbm.at[pl.ds(base + cb, 128)])
        pltpu.async_copy(tb_hbm.at[idx], rows_v, sem).wait()
        pltpu.sync_copy(rows_v, gb_hbm.at[pl.ds(base + cb, 128)])
        return 0

    lax.fori_loop(0, rpw // 128, chunk, 0)


def _sc_gates(tf, tb, ids_t):
    f = pl.kernel(
        _gates_body,
        out_type=[
            jax.ShapeDtypeStruct((T * B, 4 * HALF), jnp.float32),
            jax.ShapeDtypeStruct((T * B, 4 * HALF), jnp.float32),
        ],
        scratch_types=[
            pltpu.VMEM(((T * B) // NW,), jnp.int32),
            pltpu.VMEM((128, 4 * HALF), jnp.float32),
            pltpu.SemaphoreType.DMA,
        ],
        **_sc_mesh(),
    )
    return f(tf, tb, ids_t)


def _prep_body(adjf_hbm, adjb_hbm, nodes_hbm, fif_hbm, fib_hbm, stf_hbm, stb_hbm,
               idxn_v, rows_v, fidx_v, st_v, sem):
    wid = lax.axis_index("s") * 2 + lax.axis_index("c")
    npw = M // NW
    base = wid * npw
    pltpu.sync_copy(nodes_hbm.at[pl.ds(base, npw)], idxn_v)
    lane = jax.lax.iota(jnp.int32, 16)

    def one_dir(adj_hbm, fi_hbm, st_hbm):
        for c in range(npw // 128):
            pltpu.async_copy(
                adj_hbm.at[idxn_v.at[pl.ds(c * 128, 128)]],
                rows_v.at[pl.ds(c * 128, 128)], sem).wait()

        def mk_flat(it, _):
            q = it * 16 + lane
            v = plsc.load_gather(rows_v, [q // SAMPLE, q % SAMPLE])
            fidx_v[pl.ds(it * 16, 16)] = v
            return 0

        lax.fori_loop(0, (npw * SAMPLE) // 16, mk_flat, 0)

        for j in range(SAMPLE):
            jv = jnp.full((16,), j, jnp.int32)

            def mk_st(it, _):
                r = it * 16 + lane
                st_v[j, pl.ds(it * 16, 16)] = plsc.load_gather(rows_v, [r, jv])
                return 0

            lax.fori_loop(0, npw // 16, mk_st, 0)

        pltpu.sync_copy(fidx_v, fi_hbm.at[pl.ds(base * SAMPLE, npw * SAMPLE)])
        for j in range(SAMPLE):
            pltpu.sync_copy(st_v.at[j], st_hbm.at[j, pl.ds(base, npw)])

    one_dir(adjf_hbm, fif_hbm, stf_hbm)
    one_dir(adjb_hbm, fib_hbm, stb_hbm)


def _sc_prep(adjf, adjb, nodes):
    npw = M // NW
    f = pl.kernel(
        _prep_body,
        out_type=[
            jax.ShapeDtypeStruct((M * SAMPLE,), jnp.int32),
            jax.ShapeDtypeStruct((M * SAMPLE,), jnp.int32),
            jax.ShapeDtypeStruct((SAMPLE, M), jnp.int32),
            jax.ShapeDtypeStruct((SAMPLE, M), jnp.int32),
        ],
        scratch_types=[
            pltpu.VMEM((npw,), jnp.int32),
            pltpu.VMEM((npw, 128), jnp.int32),
            pltpu.VMEM((npw * SAMPLE,), jnp.int32),
            pltpu.VMEM((SAMPLE, npw), jnp.int32),
            pltpu.SemaphoreType.DMA,
        ],
        **_sc_mesh(),
    )
    return f(adjf, adjb, nodes)


def _gsum_dir(table_hbm, fi_hbm, out_hbm, fidx_v, rows_v, outb_v, sems, base):
    pltpu.sync_copy(fi_hbm.at[pl.ds(base * SAMPLE, (M // NW) * SAMPLE)], fidx_v)
    NCH = (M // NW) * SAMPLE // 80

    def start(k, u):
        pltpu.async_copy(
            table_hbm.at[fidx_v.at[pl.ds(k * 80, 80)]],
            rows_v.at[pl.ds(u * 80, 80)], sems[u])

    def wait(u):
        pltpu.make_async_copy(
            table_hbm.at[fidx_v.at[pl.ds(0, 80)]],
            rows_v.at[pl.ds(u * 80, 80)], sems[u]).wait()

    def reduce(u):
        off = u * 80
        for r in range(8):
            for d in range(8):
                acc = rows_v[off + r * SAMPLE, pl.ds(d * 16, 16)]
                for j in range(1, SAMPLE):
                    acc = acc + rows_v[off + r * SAMPLE + j, pl.ds(d * 16, 16)]
                outb_v[u * 8 + r, pl.ds(d * 16, 16)] = acc

    for u in range(4):
        start(u, u)

    NQ = NCH // 4

    def quad(c4, _):
        k = c4 * 4
        for u in range(4):
            wait(u)
            reduce(u)

            @pl.when(c4 < NQ - 1)
            def _():
                start(k + 4 + u, u)

        pltpu.sync_copy(outb_v, out_hbm.at[pl.ds(base + k * 8, 32)])
        return 0

    lax.fori_loop(0, NQ, quad, 0)


def _len_dir(st_hbm, p_v, invd_hbm, st_v, invd_v, base):
    npw = M // NW
    for j in range(SAMPLE):
        pltpu.sync_copy(st_hbm.at[j, pl.ds(base, npw)], st_v.at[j])

    def it_body(it, _):
        acc = jnp.zeros((16,), jnp.float32)
        for j in range(SAMPLE):
            idx = st_v[j, pl.ds(it * 16, 16)]
            acc = acc + plsc.load_gather(p_v, [idx])
        invd_v[pl.ds(it * 16, 16)] = 1.0 / jnp.maximum(acc, 1.0)
        return 0

    lax.fori_loop(0, npw // 16, it_body, 0)
    pltpu.sync_copy(invd_v, invd_hbm.at[pl.ds(base, npw)])


def _misc_body(table_hbm, nodes_hbm, stf_hbm, stb_hbm, p_hbm,
               h0_hbm, invf_hbm, invb_hbm,
               rows_v, st_v, invd_v, p_v, idxn_v, sem):
    wid = lax.axis_index("s") * 2 + lax.axis_index("c")
    npw = M // NW
    base = wid * npw
    pltpu.sync_copy(nodes_hbm.at[pl.ds(base, npw)], idxn_v)

    def h0_chunk(c, _):
        pltpu.async_copy(
            table_hbm.at[idxn_v.at[pl.ds(c * 128, 128)]],
            rows_v, sem).wait()
        pltpu.sync_copy(rows_v, h0_hbm.at[pl.ds(base + c * 128, 128)])
        return 0

    lax.fori_loop(0, npw // 128, h0_chunk, 0)
    pltpu.sync_copy(p_hbm, p_v)
    _len_dir(stf_hbm, p_v, invf_hbm, st_v, invd_v, base)
    _len_dir(stb_hbm, p_v, invb_hbm, st_v, invd_v, base)


def _sc_misc(table, nodes, stf, stb, p):
    npw = M // NW
    f = pl.kernel(
        _misc_body,
        out_type=[
            jax.ShapeDtypeStruct((M, H), jnp.float32),
            jax.ShapeDtypeStruct((M,), jnp.float32),
            jax.ShapeDtypeStruct((M,), jnp.float32),
        ],
        scratch_types=[
            pltpu.VMEM((128, H), jnp.float32),
            pltpu.VMEM((SAMPLE, npw), jnp.int32),
            pltpu.VMEM((npw,), jnp.float32),
            pltpu.VMEM((M,), jnp.float32),
            pltpu.VMEM((npw,), jnp.int32),
            pltpu.SemaphoreType.DMA,
        ],
        **_sc_mesh(),
    )
    return f(table, nodes, stf, stb, p)


def _layer12_body(tf_hbm, tb_hbm, fif_hbm, fib_hbm, sf_hbm, sb_hbm,
                  fidx_v, rows_v, outb_v, s0, s1, s2, s3):
    wid = lax.axis_index("s") * 2 + lax.axis_index("c")
    base = wid * (M // NW)
    sems = [s0, s1, s2, s3]
    _gsum_dir(tf_hbm, fif_hbm, sf_hbm, fidx_v, rows_v, outb_v, sems, base)
    _gsum_dir(tb_hbm, fib_hbm, sb_hbm, fidx_v, rows_v, outb_v, sems, base)


def _sc_layer12(tf, tb, fif, fib):
    f = pl.kernel(
        _layer12_body,
        out_type=[
            jax.ShapeDtypeStruct((M, H), jnp.float32),
            jax.ShapeDtypeStruct((M, H), jnp.float32),
        ],
        scratch_types=[
            pltpu.VMEM(((M // NW) * SAMPLE,), jnp.int32),
            pltpu.VMEM((320, H), jnp.float32),
            pltpu.VMEM((32, H), jnp.float32),
            pltpu.SemaphoreType.DMA,
            pltpu.SemaphoreType.DMA,
            pltpu.SemaphoreType.DMA,
            pltpu.SemaphoreType.DMA,
        ],
        **_sc_mesh(),
    )
    return f(tf, tb, fif, fib)


def _p_body(x_ref, p_ref):
    s = jnp.sum(jax.nn.relu(x_ref[...]), axis=1, keepdims=True)
    p_ref[...] = (s > 0).astype(jnp.float32)


def _p_kernel(table):
    R = 2048
    return pl.pallas_call(
        _p_body,
        grid=(M // R,),
        in_specs=[pl.BlockSpec((R, H), lambda i: (i, 0))],
        out_specs=pl.BlockSpec((R, 1), lambda i: (i, 0)),
        out_shape=jax.ShapeDtypeStruct((M, 1), jnp.float32),
    )(table)


def kernel(fw_adj_info, bw_adj_info, feature_info, batch_nodes, batch_wordlen,
           emb, lstm_params, padding_vector, fw_agg, bw_agg):
    Bz, Nn = batch_nodes.shape

    ids = feature_info[:-1, :].reshape(Bz, T)
    ids_t = ids.T.reshape(-1)

    (Wih_f, Whh_f, bih_f, bhh_f) = lstm_params[0]
    (Wih_b, Whh_b, bih_b, bhh_b) = lstm_params[1]
    tf, tb = _premul(
        emb, Wih_f.T, Wih_b.T,
        (bih_f + bhh_f)[None, :], (bih_b + bhh_b)[None, :])
    gf, gb = _sc_gates(tf, tb, ids_t)
    gf = gf.reshape(T, B, 4 * HALF)
    gb = gb.reshape(T, B, 4 * HALF)

    z = jnp.zeros((HALF, 4 * HALF), jnp.float32)
    W2 = jnp.concatenate([
        jnp.concatenate([Whh_f.T, z], axis=1),
        jnp.concatenate([z, Whh_b.T], axis=1)], axis=0)
    hsf, hsb, htf, htb = _bilstm(gf, gb, W2)

    output_vector = jnp.concatenate([hsf, hsb], axis=-1).transpose(1, 0, 2)
    ht_view = jnp.stack([htf, htb], axis=0).reshape(Bz, H)

    table = output_vector.reshape(-1, H)

    nodes = batch_nodes.reshape(-1)
    adjf = jnp.pad(fw_adj_info, ((0, 0), (0, 128 - MAXDEG)))
    adjb = jnp.pad(bw_adj_info, ((0, 0), (0, 128 - MAXDEG)))
    fif, fib, stf, stb = _sc_prep(adjf, adjb, nodes)

    p = _p_kernel(table).reshape(-1)

    sum_f, sum_b = _sc_layer12(table, table, fif, fib)
    h0, invdf, invdb = _sc_misc(table, nodes, stf, stb, p)
    invdf = invdf[:, None]
    invdb = invdb[:, None]

    fw_hidden = h0
    bw_hidden = h0
    for layer in range(LAYERS):
        if layer > 0:
            sum_f, sum_b = _sc_layer12(fw_hidden, bw_hidden, fif, fib)
        Wf, bf = fw_agg[layer]
        Wb, bb = bw_agg[layer]
        fw_hidden, bw_hidden = _aggregate(
            fw_hidden, sum_f, invdf, bw_hidden, sum_b, invdb,
            Wf[:, :H].T, Wf[:, H:].T, bf[None, :],
            Wb[:, :H].T, Wb[:, H:].T, bb[None, :])

    hidden = jnp.concatenate(
        [fw_hidden.reshape(Bz, Nn, H), bw_hidden.reshape(Bz, Nn, H)], axis=2)
    return (output_vector, ht_view, hidden)


# speedup vs baseline: 1.0599x; 1.0599x over previous
import functools

import jax
import jax.numpy as jnp
from jax import lax
from jax.experimental import pallas as pl
from jax.experimental.pallas import tpu as pltpu
from jax.experimental.pallas import tpu_sc as plsc

def _sc_mesh():
    return dict(
        mesh=plsc.VectorSubcoreMesh(core_axis_name="c", subcore_axis_name="s"),
        compiler_params=pltpu.CompilerParams(needs_layout_passes=False),
    )
NW = 32

B = 32
N = 512
H = 128
HALF = 64
EMB_DIM = 300
MAXDEG = 20
SAMPLE = 10
LAYERS = 3
T = 1024
M = B * N
CT = 128
NTC = T // CT


_VR = 2000


def _premul_body(x_ref, wf_ref, wb_ref, bf_ref, bb_ref, tf_ref, tb_ref):
    x = x_ref[...]
    tf_ref[...] = jnp.dot(x, wf_ref[...], preferred_element_type=jnp.float32) + bf_ref[...]
    tb_ref[...] = jnp.dot(x, wb_ref[...], preferred_element_type=jnp.float32) + bb_ref[...]


def _premul(emb, Wf, Wb, bf, bb):
    V = emb.shape[0]
    grid = (V // _VR,)
    return pl.pallas_call(
        _premul_body,
        grid=grid,
        in_specs=[
            pl.BlockSpec((_VR, EMB_DIM), lambda i: (i, 0)),
            pl.BlockSpec((EMB_DIM, 4 * HALF), lambda i: (0, 0)),
            pl.BlockSpec((EMB_DIM, 4 * HALF), lambda i: (0, 0)),
            pl.BlockSpec((1, 4 * HALF), lambda i: (0, 0)),
            pl.BlockSpec((1, 4 * HALF), lambda i: (0, 0)),
        ],
        out_specs=[
            pl.BlockSpec((_VR, 4 * HALF), lambda i: (i, 0)),
            pl.BlockSpec((_VR, 4 * HALF), lambda i: (i, 0)),
        ],
        out_shape=[
            jax.ShapeDtypeStruct((V, 4 * HALF), jnp.float32),
            jax.ShapeDtypeStruct((V, 4 * HALF), jnp.float32),
        ],
    )(emb, Wf, Wb, bf, bb)


def _lstm_body(gf_ref, gb_ref, w2_ref,
               hsf_ref, hsb_ref, htf_ref, htb_ref,
               h2_ref, cf_ref, cb_ref):
    i = pl.program_id(0)

    @pl.when(i == 0)
    def _():
        h2_ref[...] = jnp.zeros_like(h2_ref)
        cf_ref[...] = jnp.zeros_like(cf_ref)
        cb_ref[...] = jnp.zeros_like(cb_ref)

    def gates_math(g, c):
        sg = jax.nn.sigmoid(g[:, 0:2 * HALF])
        gg = jnp.tanh(g[:, 2 * HALF:3 * HALF])
        og = jax.nn.sigmoid(g[:, 3 * HALF:4 * HALF])
        c2 = sg[:, HALF:2 * HALF] * c + sg[:, 0:HALF] * gg
        h2 = og * jnp.tanh(c2)
        return h2, c2

    def step(t, carry):
        hf, cf, hb, cb = carry
        tr = CT - 1 - t
        gf = gf_ref[t] + jnp.dot(hf, w2_ref[0:HALF, 0:4 * HALF],
                                 preferred_element_type=jnp.float32)
        gb = gb_ref[tr] + jnp.dot(hb, w2_ref[HALF:2 * HALF, 4 * HALF:8 * HALF],
                                  preferred_element_type=jnp.float32)
        h2f, c2f = gates_math(gf, cf)
        h2b, c2b = gates_math(gb, cb)
        hsf_ref[t] = h2f
        hsb_ref[tr] = h2b
        return h2f, c2f, h2b, c2b

    h0 = h2_ref[...]
    hf, cf, hb, cb = lax.fori_loop(
        0, CT, step,
        (h0[:, 0:HALF], cf_ref[...], h0[:, HALF:2 * HALF], cb_ref[...]),
        unroll=4)
    h2 = jnp.concatenate([hf, hb], axis=-1)
    h2_ref[...] = h2
    cf_ref[...] = cf
    cb_ref[...] = cb

    @pl.when(i == NTC - 1)
    def _():
        htf_ref[...] = h2[:, 0:HALF]
        htb_ref[...] = h2[:, HALF:2 * HALF]


def _bilstm(gf, gb, W2):
    return pl.pallas_call(
        _lstm_body,
        grid=(NTC,),
        in_specs=[
            pl.BlockSpec((CT, B, 4 * HALF), lambda i: (i, 0, 0)),
            pl.BlockSpec((CT, B, 4 * HALF), lambda i: (NTC - 1 - i, 0, 0)),
            pl.BlockSpec((2 * HALF, 8 * HALF), lambda i: (0, 0)),
        ],
        out_specs=[
            pl.BlockSpec((CT, B, HALF), lambda i: (i, 0, 0)),
            pl.BlockSpec((CT, B, HALF), lambda i: (NTC - 1 - i, 0, 0)),
            pl.BlockSpec((B, HALF), lambda i: (0, 0)),
            pl.BlockSpec((B, HALF), lambda i: (0, 0)),
        ],
        out_shape=[
            jax.ShapeDtypeStruct((T, B, HALF), jnp.float32),
            jax.ShapeDtypeStruct((T, B, HALF), jnp.float32),
            jax.ShapeDtypeStruct((B, HALF), jnp.float32),
            jax.ShapeDtypeStruct((B, HALF), jnp.float32),
        ],
        scratch_shapes=[
            pltpu.VMEM((B, 2 * HALF), jnp.float32),
            pltpu.VMEM((B, HALF), jnp.float32),
            pltpu.VMEM((B, HALF), jnp.float32),
        ],
    )(gf, gb, W2)


def _agg_body(hf_ref, sf_ref, df_ref, hb_ref, sb_ref, db_ref,
              w1f_ref, w2f_ref, bf_ref, w1b_ref, w2b_ref, bb_ref,
              of_ref, ob_ref):
    nf = sf_ref[...] * df_ref[...]
    nb = sb_ref[...] * db_ref[...]
    of_ref[...] = jax.nn.relu(
        jnp.dot(hf_ref[...], w1f_ref[...], preferred_element_type=jnp.float32)
        + jnp.dot(nf, w2f_ref[...], preferred_element_type=jnp.float32) + bf_ref[...])
    ob_ref[...] = jax.nn.relu(
        jnp.dot(hb_ref[...], w1b_ref[...], preferred_element_type=jnp.float32)
        + jnp.dot(nb, w2b_ref[...], preferred_element_type=jnp.float32) + bb_ref[...])


def _aggregate(hf, sumf, invdf, hb, sumb, invdb, w1f, w2f, bf, w1b, w2b, bb):
    R = 2048
    grid = (M // R,)
    row = lambda i: (i, 0)
    fixed = lambda i: (0, 0)
    return pl.pallas_call(
        _agg_body,
        grid=grid,
        in_specs=[
            pl.BlockSpec((R, H), row), pl.BlockSpec((R, H), row), pl.BlockSpec((R, 1), row),
            pl.BlockSpec((R, H), row), pl.BlockSpec((R, H), row), pl.BlockSpec((R, 1), row),
            pl.BlockSpec((H, H), fixed), pl.BlockSpec((H, H), fixed), pl.BlockSpec((1, H), fixed),
            pl.BlockSpec((H, H), fixed), pl.BlockSpec((H, H), fixed), pl.BlockSpec((1, H), fixed),
        ],
        out_specs=[pl.BlockSpec((R, H), row), pl.BlockSpec((R, H), row)],
        out_shape=[
            jax.ShapeDtypeStruct((M, H), jnp.float32),
            jax.ShapeDtypeStruct((M, H), jnp.float32),
        ],
    )(hf, sumf, invdf, hb, sumb, invdb, w1f, w2f, bf, w1b, w2b, bb)


def _gates_body(tf_hbm, tb_hbm, ids_hbm, gf_hbm, gb_hbm, idx_v, rows_v, sem):
    wid = lax.axis_index("s") * 2 + lax.axis_index("c")
    rpw = (T * B) // NW
    base = wid * rpw
    pltpu.sync_copy(ids_hbm.at[pl.ds(base, rpw)], idx_v)

    def chunk(c, _):
        cb = c * 128
        idx = idx_v.at[pl.ds(cb, 128)]
        pltpu.async_copy(tf_hbm.at[idx], rows_v, sem).wait()
        pltpu.sync_copy(rows_v, gf_hbm.at[pl.ds(base + cb, 128)])
        pltpu.async_copy(tb_hbm.at[idx], rows_v, sem).wait()
        pltpu.sync_copy(rows_v, gb_hbm.at[pl.ds(base + cb, 128)])
        return 0

    lax.fori_loop(0, rpw // 128, chunk, 0)


def _sc_gates(tf, tb, ids_t):
    f = pl.kernel(
        _gates_body,
        out_type=[
            jax.ShapeDtypeStruct((T * B, 4 * HALF), jnp.float32),
            jax.ShapeDtypeStruct((T * B, 4 * HALF), jnp.float32),
        ],
        scratch_types=[
            pltpu.VMEM(((T * B) // NW,), jnp.int32),
            pltpu.VMEM((128, 4 * HALF), jnp.float32),
            pltpu.SemaphoreType.DMA,
        ],
        **_sc_mesh(),
    )
    return f(tf, tb, ids_t)


def _prep_body(adjf_hbm, adjb_hbm, nodes_hbm, fif_hbm, fib_hbm, stf_hbm, stb_hbm,
               idxn_v, rows_v, fidx_v, st_v, sem):
    wid = lax.axis_index("s") * 2 + lax.axis_index("c")
    npw = M // NW
    base = wid * npw
    pltpu.sync_copy(nodes_hbm.at[pl.ds(base, npw)], idxn_v)
    lane = jax.lax.iota(jnp.int32, 16)

    def one_dir(adj_hbm, fi_hbm, st_hbm):
        for c in range(npw // 128):
            pltpu.async_copy(
                adj_hbm.at[idxn_v.at[pl.ds(c * 128, 128)]],
                rows_v.at[pl.ds(c * 128, 128)], sem).wait()

        def mk_flat(it, _):
            q = it * 16 + lane
            v = plsc.load_gather(rows_v, [q // SAMPLE, q % SAMPLE])
            fidx_v[pl.ds(it * 16, 16)] = v
            return 0

        lax.fori_loop(0, (npw * SAMPLE) // 16, mk_flat, 0)

        for j in range(SAMPLE):
            jv = jnp.full((16,), j, jnp.int32)

            def mk_st(it, _):
                r = it * 16 + lane
                st_v[j, pl.ds(it * 16, 16)] = plsc.load_gather(rows_v, [r, jv])
                return 0

            lax.fori_loop(0, npw // 16, mk_st, 0)

        pltpu.sync_copy(fidx_v, fi_hbm.at[pl.ds(base * SAMPLE, npw * SAMPLE)])
        for j in range(SAMPLE):
            pltpu.sync_copy(st_v.at[j], st_hbm.at[j, pl.ds(base, npw)])

    one_dir(adjf_hbm, fif_hbm, stf_hbm)
    one_dir(adjb_hbm, fib_hbm, stb_hbm)


def _sc_prep(adjf, adjb, nodes):
    npw = M // NW
    f = pl.kernel(
        _prep_body,
        out_type=[
            jax.ShapeDtypeStruct((M * SAMPLE,), jnp.int32),
            jax.ShapeDtypeStruct((M * SAMPLE,), jnp.int32),
            jax.ShapeDtypeStruct((SAMPLE, M), jnp.int32),
            jax.ShapeDtypeStruct((SAMPLE, M), jnp.int32),
        ],
        scratch_types=[
            pltpu.VMEM((npw,), jnp.int32),
            pltpu.VMEM((npw, 128), jnp.int32),
            pltpu.VMEM((npw * SAMPLE,), jnp.int32),
            pltpu.VMEM((SAMPLE, npw), jnp.int32),
            pltpu.SemaphoreType.DMA,
        ],
        **_sc_mesh(),
    )
    return f(adjf, adjb, nodes)


def _gsum_dir(table_hbm, fi_hbm, out_hbm, fidx_v, rows_v, outb_v, sems, base):
    pltpu.sync_copy(fi_hbm.at[pl.ds(base * SAMPLE, (M // NW) * SAMPLE)], fidx_v)
    NCH = (M // NW) * SAMPLE // 80

    def start(k, u):
        pltpu.async_copy(
            table_hbm.at[fidx_v.at[pl.ds(k * 80, 80)]],
            rows_v.at[pl.ds(u * 80, 80)], sems[u])

    def wait(u):
        pltpu.make_async_copy(
            table_hbm.at[fidx_v.at[pl.ds(0, 80)]],
            rows_v.at[pl.ds(u * 80, 80)], sems[u]).wait()

    def reduce(u):
        off = u * 80
        for r in range(8):
            for d in range(8):
                acc = rows_v[off + r * SAMPLE, pl.ds(d * 16, 16)]
                for j in range(1, SAMPLE):
                    acc = acc + rows_v[off + r * SAMPLE + j, pl.ds(d * 16, 16)]
                outb_v[u * 8 + r, pl.ds(d * 16, 16)] = acc

    for u in range(4):
        start(u, u)

    NQ = NCH // 4

    def quad(c4, _):
        k = c4 * 4
        for u in range(4):
            wait(u)
            reduce(u)

            @pl.when(c4 < NQ - 1)
            def _():
                start(k + 4 + u, u)

        pltpu.sync_copy(outb_v, out_hbm.at[pl.ds(base + k * 8, 32)])
        return 0

    lax.fori_loop(0, NQ, quad, 0)


def _len_dir(st_hbm, p_v, invd_hbm, st_v, invd_v, base):
    npw = M // NW
    for j in range(SAMPLE):
        pltpu.sync_copy(st_hbm.at[j, pl.ds(base, npw)], st_v.at[j])

    def it_body(it, _):
        acc = jnp.zeros((16,), jnp.float32)
        for j in range(SAMPLE):
            idx = st_v[j, pl.ds(it * 16, 16)]
            acc = acc + plsc.load_gather(p_v, [idx])
        invd_v[pl.ds(it * 16, 16)] = 1.0 / jnp.maximum(acc, 1.0)
        return 0

    lax.fori_loop(0, npw // 16, it_body, 0)
    pltpu.sync_copy(invd_v, invd_hbm.at[pl.ds(base, npw)])


def _misc_body(table_hbm, nodes_hbm, stf_hbm, stb_hbm, p_hbm,
               h0_hbm, invf_hbm, invb_hbm,
               rows_v, st_v, invd_v, p_v, idxn_v, sem):
    wid = lax.axis_index("s") * 2 + lax.axis_index("c")
    npw = M // NW
    base = wid * npw
    pltpu.sync_copy(nodes_hbm.at[pl.ds(base, npw)], idxn_v)

    def h0_chunk(c, _):
        pltpu.async_copy(
            table_hbm.at[idxn_v.at[pl.ds(c * 128, 128)]],
            rows_v, sem).wait()
        pltpu.sync_copy(rows_v, h0_hbm.at[pl.ds(base + c * 128, 128)])
        return 0

    lax.fori_loop(0, npw // 128, h0_chunk, 0)
    pltpu.sync_copy(p_hbm, p_v)
    _len_dir(stf_hbm, p_v, invf_hbm, st_v, invd_v, base)
    _len_dir(stb_hbm, p_v, invb_hbm, st_v, invd_v, base)


def _sc_misc(table, nodes, stf, stb, p):
    npw = M // NW
    f = pl.kernel(
        _misc_body,
        out_type=[
            jax.ShapeDtypeStruct((M, H), jnp.float32),
            jax.ShapeDtypeStruct((M,), jnp.float32),
            jax.ShapeDtypeStruct((M,), jnp.float32),
        ],
        scratch_types=[
            pltpu.VMEM((128, H), jnp.float32),
            pltpu.VMEM((SAMPLE, npw), jnp.int32),
            pltpu.VMEM((npw,), jnp.float32),
            pltpu.VMEM((M,), jnp.float32),
            pltpu.VMEM((npw,), jnp.int32),
            pltpu.SemaphoreType.DMA,
        ],
        **_sc_mesh(),
    )
    return f(table, nodes, stf, stb, p)


def _layer12_body(tf_hbm, tb_hbm, fif_hbm, fib_hbm, sf_hbm, sb_hbm,
                  fidx_v, rows_v, outb_v, s0, s1, s2, s3):
    wid = lax.axis_index("s") * 2 + lax.axis_index("c")
    base = wid * (M // NW)
    sems = [s0, s1, s2, s3]
    _gsum_dir(tf_hbm, fif_hbm, sf_hbm, fidx_v, rows_v, outb_v, sems, base)
    _gsum_dir(tb_hbm, fib_hbm, sb_hbm, fidx_v, rows_v, outb_v, sems, base)


def _sc_layer12(tf, tb, fif, fib):
    f = pl.kernel(
        _layer12_body,
        out_type=[
            jax.ShapeDtypeStruct((M, H), jnp.float32),
            jax.ShapeDtypeStruct((M, H), jnp.float32),
        ],
        scratch_types=[
            pltpu.VMEM(((M // NW) * SAMPLE,), jnp.int32),
            pltpu.VMEM((320, H), jnp.float32),
            pltpu.VMEM((32, H), jnp.float32),
            pltpu.SemaphoreType.DMA,
            pltpu.SemaphoreType.DMA,
            pltpu.SemaphoreType.DMA,
            pltpu.SemaphoreType.DMA,
        ],
        **_sc_mesh(),
    )
    return f(tf, tb, fif, fib)


def _p_body(x_ref, p_ref):
    s = jnp.sum(jax.nn.relu(x_ref[...]), axis=1, keepdims=True)
    p_ref[...] = (s > 0).astype(jnp.float32)


def _p_kernel(table):
    R = 2048
    return pl.pallas_call(
        _p_body,
        grid=(M // R,),
        in_specs=[pl.BlockSpec((R, H), lambda i: (i, 0))],
        out_specs=pl.BlockSpec((R, 1), lambda i: (i, 0)),
        out_shape=jax.ShapeDtypeStruct((M, 1), jnp.float32),
    )(table)


def kernel(fw_adj_info, bw_adj_info, feature_info, batch_nodes, batch_wordlen,
           emb, lstm_params, padding_vector, fw_agg, bw_agg):
    Bz, Nn = batch_nodes.shape

    ids = feature_info[:-1, :].reshape(Bz, T)
    ids_t = ids.T.reshape(-1)

    (Wih_f, Whh_f, bih_f, bhh_f) = lstm_params[0]
    (Wih_b, Whh_b, bih_b, bhh_b) = lstm_params[1]
    tf, tb = _premul(
        emb, Wih_f.T, Wih_b.T,
        (bih_f + bhh_f)[None, :], (bih_b + bhh_b)[None, :])
    gf, gb = _sc_gates(tf, tb, ids_t)
    gf = gf.reshape(T, B, 4 * HALF)
    gb = gb.reshape(T, B, 4 * HALF)

    z = jnp.zeros((HALF, 4 * HALF), jnp.float32)
    W2 = jnp.concatenate([
        jnp.concatenate([Whh_f.T, z], axis=1),
        jnp.concatenate([z, Whh_b.T], axis=1)], axis=0)
    hsf, hsb, htf, htb = _bilstm(gf, gb, W2)

    output_vector = jnp.concatenate([hsf, hsb], axis=-1).transpose(1, 0, 2)
    ht_view = jnp.stack([htf, htb], axis=0).reshape(Bz, H)

    table = output_vector.reshape(-1, H)

    nodes = batch_nodes.reshape(-1)
    adjf = jnp.pad(fw_adj_info, ((0, 0), (0, 128 - MAXDEG)))
    adjb = jnp.pad(bw_adj_info, ((0, 0), (0, 128 - MAXDEG)))
    fif, fib, stf, stb = _sc_prep(adjf, adjb, nodes)

    p = _p_kernel(table).reshape(-1)

    sum_f, sum_b = _sc_layer12(table, table, fif, fib)
    h0, invdf, invdb = _sc_misc(table, nodes, stf, stb, p)
    invdf = invdf[:, None]
    invdb = invdb[:, None]

    fw_hidden = h0
    bw_hidden = h0
    for layer in range(LAYERS):
        if layer > 0:
            sum_f, sum_b = _sc_layer12(fw_hidden, bw_hidden, fif, fib)
        Wf, bf = fw_agg[layer]
        Wb, bb = bw_agg[layer]
        fw_hidden, bw_hidden = _aggregate(
            fw_hidden, sum_f, invdf, bw_hidden, sum_b, invdb,
            Wf[:, :H].T, Wf[:, H:].T, bf[None, :],
            Wb[:, :H].T, Wb[:, H:].T, bb[None, :])

    hidden = jnp.concatenate(
        [fw_hidden.reshape(Bz, Nn, H), bw_hidden.reshape(Bz, Nn, H)], axis=2)
    return (output_vector, ht_view, hidden)


# speedup vs baseline: 1.0643x; 1.0041x over previous
import functools

import jax
import jax.numpy as jnp
from jax import lax
from jax.experimental import pallas as pl
from jax.experimental.pallas import tpu as pltpu
from jax.experimental.pallas import tpu_sc as plsc

def _sc_mesh():
    return dict(
        mesh=plsc.VectorSubcoreMesh(core_axis_name="c", subcore_axis_name="s"),
        compiler_params=pltpu.CompilerParams(needs_layout_passes=False),
    )
NW = 32

B = 32
N = 512
H = 128
HALF = 64
EMB_DIM = 300
MAXDEG = 20
SAMPLE = 10
LAYERS = 3
T = 1024
M = B * N
CT = 128
NTC = T // CT


_VR = 2000


def _premul_body(x_ref, wf_ref, wb_ref, bf_ref, bb_ref, tf_ref, tb_ref):
    x = x_ref[...]
    tf_ref[...] = jnp.dot(x, wf_ref[...], preferred_element_type=jnp.float32) + bf_ref[...]
    tb_ref[...] = jnp.dot(x, wb_ref[...], preferred_element_type=jnp.float32) + bb_ref[...]


def _premul(emb, Wf, Wb, bf, bb):
    V = emb.shape[0]
    grid = (V // _VR,)
    return pl.pallas_call(
        _premul_body,
        grid=grid,
        in_specs=[
            pl.BlockSpec((_VR, EMB_DIM), lambda i: (i, 0)),
            pl.BlockSpec((EMB_DIM, 4 * HALF), lambda i: (0, 0)),
            pl.BlockSpec((EMB_DIM, 4 * HALF), lambda i: (0, 0)),
            pl.BlockSpec((1, 4 * HALF), lambda i: (0, 0)),
            pl.BlockSpec((1, 4 * HALF), lambda i: (0, 0)),
        ],
        out_specs=[
            pl.BlockSpec((_VR, 4 * HALF), lambda i: (i, 0)),
            pl.BlockSpec((_VR, 4 * HALF), lambda i: (i, 0)),
        ],
        out_shape=[
            jax.ShapeDtypeStruct((V, 4 * HALF), jnp.float32),
            jax.ShapeDtypeStruct((V, 4 * HALF), jnp.float32),
        ],
    )(emb, Wf, Wb, bf, bb)


def _lstm_body(gf_ref, gb_ref, w2_ref,
               hsf_ref, hsb_ref, htf_ref, htb_ref,
               h2_ref, cf_ref, cb_ref):
    i = pl.program_id(0)

    @pl.when(i == 0)
    def _():
        h2_ref[...] = jnp.zeros_like(h2_ref)
        cf_ref[...] = jnp.zeros_like(cf_ref)
        cb_ref[...] = jnp.zeros_like(cb_ref)

    def gates_math(g, c):
        sg = jax.nn.sigmoid(g[:, 0:2 * HALF])
        gg = jnp.tanh(g[:, 2 * HALF:3 * HALF])
        og = jax.nn.sigmoid(g[:, 3 * HALF:4 * HALF])
        c2 = sg[:, HALF:2 * HALF] * c + sg[:, 0:HALF] * gg
        h2 = og * jnp.tanh(c2)
        return h2, c2

    def step(t, carry):
        hf, cf, hb, cb = carry
        tr = CT - 1 - t
        gf = gf_ref[t] + jnp.dot(hf, w2_ref[0:HALF, 0:4 * HALF],
                                 preferred_element_type=jnp.float32)
        gb = gb_ref[tr] + jnp.dot(hb, w2_ref[HALF:2 * HALF, 4 * HALF:8 * HALF],
                                  preferred_element_type=jnp.float32)
        h2f, c2f = gates_math(gf, cf)
        h2b, c2b = gates_math(gb, cb)
        hsf_ref[:, t, :] = h2f
        hsb_ref[:, tr, :] = h2b
        return h2f, c2f, h2b, c2b

    h0 = h2_ref[...]
    hf, cf, hb, cb = lax.fori_loop(
        0, CT, step,
        (h0[:, 0:HALF], cf_ref[...], h0[:, HALF:2 * HALF], cb_ref[...]),
        unroll=4)
    h2 = jnp.concatenate([hf, hb], axis=-1)
    h2_ref[...] = h2
    cf_ref[...] = cf
    cb_ref[...] = cb

    @pl.when(i == NTC - 1)
    def _():
        htf_ref[...] = h2[:, 0:HALF]
        htb_ref[...] = h2[:, HALF:2 * HALF]


def _bilstm(gf, gb, W2):
    return pl.pallas_call(
        _lstm_body,
        grid=(NTC,),
        in_specs=[
            pl.BlockSpec((CT, B, 4 * HALF), lambda i: (i, 0, 0)),
            pl.BlockSpec((CT, B, 4 * HALF), lambda i: (NTC - 1 - i, 0, 0)),
            pl.BlockSpec((2 * HALF, 8 * HALF), lambda i: (0, 0)),
        ],
        out_specs=[
            pl.BlockSpec((B, CT, HALF), lambda i: (0, i, 0)),
            pl.BlockSpec((B, CT, HALF), lambda i: (0, NTC - 1 - i, 0)),
            pl.BlockSpec((B, HALF), lambda i: (0, 0)),
            pl.BlockSpec((B, HALF), lambda i: (0, 0)),
        ],
        out_shape=[
            jax.ShapeDtypeStruct((B, T, HALF), jnp.float32),
            jax.ShapeDtypeStruct((B, T, HALF), jnp.float32),
            jax.ShapeDtypeStruct((B, HALF), jnp.float32),
            jax.ShapeDtypeStruct((B, HALF), jnp.float32),
        ],
        scratch_shapes=[
            pltpu.VMEM((B, 2 * HALF), jnp.float32),
            pltpu.VMEM((B, HALF), jnp.float32),
            pltpu.VMEM((B, HALF), jnp.float32),
        ],
    )(gf, gb, W2)


def _agg_body(h_ref, s_ref, d_ref, w1_ref, w2_ref, b_ref, o_ref):
    n = s_ref[...] * d_ref[...]
    o_ref[...] = jax.nn.relu(
        jnp.dot(h_ref[...], w1_ref[...], preferred_element_type=jnp.float32)
        + jnp.dot(n, w2_ref[...], preferred_element_type=jnp.float32) + b_ref[...])


def _aggregate(h, summ, invd, w1, w2, b):
    R = 2048
    grid = (M // R,)
    row = lambda i: (i, 0)
    fixed = lambda i: (0, 0)
    return pl.pallas_call(
        _agg_body,
        grid=grid,
        in_specs=[
            pl.BlockSpec((R, H), row), pl.BlockSpec((R, H), row), pl.BlockSpec((R, 1), row),
            pl.BlockSpec((H, H), fixed), pl.BlockSpec((H, H), fixed), pl.BlockSpec((1, H), fixed),
        ],
        out_specs=pl.BlockSpec((R, H), row),
        out_shape=jax.ShapeDtypeStruct((M, H), jnp.float32),
    )(h, summ, invd, w1, w2, b)


def _gates_body(tf_hbm, tb_hbm, ids_hbm, gf_hbm, gb_hbm, idx_v, rows_v, sem):
    wid = lax.axis_index("s") * 2 + lax.axis_index("c")
    rpw = (T * B) // NW
    base = wid * rpw
    pltpu.sync_copy(ids_hbm.at[pl.ds(base, rpw)], idx_v)

    def chunk(c, _):
        cb = c * 128
        idx = idx_v.at[pl.ds(cb, 128)]
        pltpu.async_copy(tf_hbm.at[idx], rows_v, sem).wait()
        pltpu.sync_copy(rows_v, gf_hbm.at[pl.ds(base + cb, 128)])
        pltpu.async_copy(tb_hbm.at[idx], rows_v, sem).wait()
        pltpu.sync_copy(rows_v, gb_hbm.at[pl.ds(base + cb, 128)])
        return 0

    lax.fori_loop(0, rpw // 128, chunk, 0)


def _sc_gates(tf, tb, ids_t):
    f = pl.kernel(
        _gates_body,
        out_type=[
            jax.ShapeDtypeStruct((T * B, 4 * HALF), jnp.float32),
            jax.ShapeDtypeStruct((T * B, 4 * HALF), jnp.float32),
        ],
        scratch_types=[
            pltpu.VMEM(((T * B) // NW,), jnp.int32),
            pltpu.VMEM((128, 4 * HALF), jnp.float32),
            pltpu.SemaphoreType.DMA,
        ],
        **_sc_mesh(),
    )
    return f(tf, tb, ids_t)


def _prep_body(adjf_hbm, adjb_hbm, nodes_hbm, fif_hbm, fib_hbm, stf_hbm, stb_hbm,
               idxn_v, rows_v, fidx_v, st_v, sem):
    wid = lax.axis_index("s") * 2 + lax.axis_index("c")
    npw = M // NW
    base = wid * npw
    pltpu.sync_copy(nodes_hbm.at[pl.ds(base, npw)], idxn_v)
    lane = jax.lax.iota(jnp.int32, 16)

    def one_dir(adj_hbm, fi_hbm, st_hbm):
        for c in range(npw // 128):
            pltpu.async_copy(
                adj_hbm.at[idxn_v.at[pl.ds(c * 128, 128)]],
                rows_v.at[pl.ds(c * 128, 128)], sem).wait()

        def mk_flat(it, _):
            q = it * 16 + lane
            v = plsc.load_gather(rows_v, [q // SAMPLE, q % SAMPLE])
            fidx_v[pl.ds(it * 16, 16)] = v
            return 0

        lax.fori_loop(0, (npw * SAMPLE) // 16, mk_flat, 0)

        for j in range(SAMPLE):
            jv = jnp.full((16,), j, jnp.int32)

            def mk_st(it, _):
                r = it * 16 + lane
                st_v[j, pl.ds(it * 16, 16)] = plsc.load_gather(rows_v, [r, jv])
                return 0

            lax.fori_loop(0, npw // 16, mk_st, 0)

        pltpu.sync_copy(fidx_v, fi_hbm.at[pl.ds(base * SAMPLE, npw * SAMPLE)])
        for j in range(SAMPLE):
            pltpu.sync_copy(st_v.at[j], st_hbm.at[j, pl.ds(base, npw)])

    one_dir(adjf_hbm, fif_hbm, stf_hbm)
    one_dir(adjb_hbm, fib_hbm, stb_hbm)


def _sc_prep(adjf, adjb, nodes):
    npw = M // NW
    f = pl.kernel(
        _prep_body,
        out_type=[
            jax.ShapeDtypeStruct((M * SAMPLE,), jnp.int32),
            jax.ShapeDtypeStruct((M * SAMPLE,), jnp.int32),
            jax.ShapeDtypeStruct((SAMPLE, M), jnp.int32),
            jax.ShapeDtypeStruct((SAMPLE, M), jnp.int32),
        ],
        scratch_types=[
            pltpu.VMEM((npw,), jnp.int32),
            pltpu.VMEM((npw, 128), jnp.int32),
            pltpu.VMEM((npw * SAMPLE,), jnp.int32),
            pltpu.VMEM((SAMPLE, npw), jnp.int32),
            pltpu.SemaphoreType.DMA,
        ],
        **_sc_mesh(),
    )
    return f(adjf, adjb, nodes)


def _gsum_dir(table_hbm, fi_hbm, out_hbm, fidx_v, rows_v, outb_v, sems, base):
    pltpu.sync_copy(fi_hbm.at[pl.ds(base * SAMPLE, (M // NW) * SAMPLE)], fidx_v)
    NCH = (M // NW) * SAMPLE // 80

    def start(k, u):
        pltpu.async_copy(
            table_hbm.at[fidx_v.at[pl.ds(k * 80, 80)]],
            rows_v.at[pl.ds(u * 80, 80)], sems[u])

    def wait(u):
        pltpu.make_async_copy(
            table_hbm.at[fidx_v.at[pl.ds(0, 80)]],
            rows_v.at[pl.ds(u * 80, 80)], sems[u]).wait()

    def reduce(u):
        off = u * 80
        for r in range(8):
            for d in range(8):
                acc = rows_v[off + r * SAMPLE, pl.ds(d * 16, 16)]
                for j in range(1, SAMPLE):
                    acc = acc + rows_v[off + r * SAMPLE + j, pl.ds(d * 16, 16)]
                outb_v[u * 8 + r, pl.ds(d * 16, 16)] = acc

    for u in range(4):
        start(u, u)

    NQ = NCH // 4

    def quad(c4, _):
        k = c4 * 4
        for u in range(4):
            wait(u)
            reduce(u)

            @pl.when(c4 < NQ - 1)
            def _():
                start(k + 4 + u, u)

        pltpu.sync_copy(outb_v, out_hbm.at[pl.ds(base + k * 8, 32)])
        return 0

    lax.fori_loop(0, NQ, quad, 0)


def _len_dir(st_hbm, p_v, invd_hbm, st_v, invd_v, base):
    npw = M // NW
    for j in range(SAMPLE):
        pltpu.sync_copy(st_hbm.at[j, pl.ds(base, npw)], st_v.at[j])

    def it_body(it, _):
        acc = jnp.zeros((16,), jnp.float32)
        for j in range(SAMPLE):
            idx = st_v[j, pl.ds(it * 16, 16)]
            acc = acc + plsc.load_gather(p_v, [idx])
        invd_v[pl.ds(it * 16, 16)] = 1.0 / jnp.maximum(acc, 1.0)
        return 0

    lax.fori_loop(0, npw // 16, it_body, 0)
    pltpu.sync_copy(invd_v, invd_hbm.at[pl.ds(base, npw)])


def _misc_body(table_hbm, nodes_hbm, stf_hbm, stb_hbm, p_hbm,
               h0_hbm, invf_hbm, invb_hbm,
               rows_v, st_v, invd_v, p_v, idxn_v, sem):
    wid = lax.axis_index("s") * 2 + lax.axis_index("c")
    npw = M // NW
    base = wid * npw
    pltpu.sync_copy(nodes_hbm.at[pl.ds(base, npw)], idxn_v)

    def h0_chunk(c, _):
        pltpu.async_copy(
            table_hbm.at[idxn_v.at[pl.ds(c * 128, 128)]],
            rows_v, sem).wait()
        pltpu.sync_copy(rows_v, h0_hbm.at[pl.ds(base + c * 128, 128)])
        return 0

    lax.fori_loop(0, npw // 128, h0_chunk, 0)
    pltpu.sync_copy(p_hbm, p_v)
    _len_dir(stf_hbm, p_v, invf_hbm, st_v, invd_v, base)
    _len_dir(stb_hbm, p_v, invb_hbm, st_v, invd_v, base)


def _sc_misc(table, nodes, stf, stb, p):
    npw = M // NW
    f = pl.kernel(
        _misc_body,
        out_type=[
            jax.ShapeDtypeStruct((M, H), jnp.float32),
            jax.ShapeDtypeStruct((M,), jnp.float32),
            jax.ShapeDtypeStruct((M,), jnp.float32),
        ],
        scratch_types=[
            pltpu.VMEM((128, H), jnp.float32),
            pltpu.VMEM((SAMPLE, npw), jnp.int32),
            pltpu.VMEM((npw,), jnp.float32),
            pltpu.VMEM((M,), jnp.float32),
            pltpu.VMEM((npw,), jnp.int32),
            pltpu.SemaphoreType.DMA,
        ],
        **_sc_mesh(),
    )
    return f(table, nodes, stf, stb, p)


def _gsum_body(t_hbm, fi_hbm, s_hbm, fidx_v, rows_v, outb_v, s0, s1, s2, s3):
    wid = lax.axis_index("s") * 2 + lax.axis_index("c")
    base = wid * (M // NW)
    _gsum_dir(t_hbm, fi_hbm, s_hbm, fidx_v, rows_v, outb_v, [s0, s1, s2, s3], base)


def _sc_gsum(table, fi):
    f = pl.kernel(
        _gsum_body,
        out_type=jax.ShapeDtypeStruct((M, H), jnp.float32),
        scratch_types=[
            pltpu.VMEM(((M // NW) * SAMPLE,), jnp.int32),
            pltpu.VMEM((320, H), jnp.float32),
            pltpu.VMEM((32, H), jnp.float32),
            pltpu.SemaphoreType.DMA,
            pltpu.SemaphoreType.DMA,
            pltpu.SemaphoreType.DMA,
            pltpu.SemaphoreType.DMA,
        ],
        **_sc_mesh(),
    )
    return f(table, fi)


def _p_body(x_ref, p_ref):
    s = jnp.sum(jax.nn.relu(x_ref[...]), axis=1, keepdims=True)
    p_ref[...] = (s > 0).astype(jnp.float32)


def _p_kernel(table):
    R = 2048
    return pl.pallas_call(
        _p_body,
        grid=(M // R,),
        in_specs=[pl.BlockSpec((R, H), lambda i: (i, 0))],
        out_specs=pl.BlockSpec((R, 1), lambda i: (i, 0)),
        out_shape=jax.ShapeDtypeStruct((M, 1), jnp.float32),
    )(table)


def kernel(fw_adj_info, bw_adj_info, feature_info, batch_nodes, batch_wordlen,
           emb, lstm_params, padding_vector, fw_agg, bw_agg):
    Bz, Nn = batch_nodes.shape

    ids = feature_info[:-1, :].reshape(Bz, T)
    ids_t = ids.T.reshape(-1)

    (Wih_f, Whh_f, bih_f, bhh_f) = lstm_params[0]
    (Wih_b, Whh_b, bih_b, bhh_b) = lstm_params[1]
    tf, tb = _premul(
        emb, Wih_f.T, Wih_b.T,
        (bih_f + bhh_f)[None, :], (bih_b + bhh_b)[None, :])
    gf, gb = _sc_gates(tf, tb, ids_t)
    gf = gf.reshape(T, B, 4 * HALF)
    gb = gb.reshape(T, B, 4 * HALF)

    z = jnp.zeros((HALF, 4 * HALF), jnp.float32)
    W2 = jnp.concatenate([
        jnp.concatenate([Whh_f.T, z], axis=1),
        jnp.concatenate([z, Whh_b.T], axis=1)], axis=0)
    hsf, hsb, htf, htb = _bilstm(gf, gb, W2)

    output_vector = jnp.concatenate([hsf, hsb], axis=-1)
    ht_view = jnp.stack([htf, htb], axis=0).reshape(Bz, H)

    table = output_vector.reshape(-1, H)

    nodes = batch_nodes.reshape(-1)
    adjf = jnp.pad(fw_adj_info, ((0, 0), (0, 128 - MAXDEG)))
    adjb = jnp.pad(bw_adj_info, ((0, 0), (0, 128 - MAXDEG)))
    fif, fib, stf, stb = _sc_prep(adjf, adjb, nodes)

    p = _p_kernel(table).reshape(-1)

    h0, invdf, invdb = _sc_misc(table, nodes, stf, stb, p)
    invdf = invdf[:, None]
    invdb = invdb[:, None]

    fw_hidden = h0
    bw_hidden = h0
    for layer in range(LAYERS):
        src_f = table if layer == 0 else fw_hidden
        src_b = table if layer == 0 else bw_hidden
        sum_f = _sc_gsum(src_f, fif)
        sum_b = _sc_gsum(src_b, fib)
        Wf, bf = fw_agg[layer]
        Wb, bb = bw_agg[layer]
        fw_hidden = _aggregate(fw_hidden, sum_f, invdf,
                               Wf[:, :H].T, Wf[:, H:].T, bf[None, :])
        bw_hidden = _aggregate(bw_hidden, sum_b, invdb,
                               Wb[:, :H].T, Wb[:, H:].T, bb[None, :])

    hidden = jnp.concatenate(
        [fw_hidden.reshape(Bz, Nn, H), bw_hidden.reshape(Bz, Nn, H)], axis=2)
    return (output_vector, ht_view, hidden)
